# Initial kernel scaffold; baseline (speedup 1.0000x reference)
import jax
import jax.numpy as jnp
from jax.experimental import pallas as pl


def kernel(pos, info_level, from_prior, domain_node_index):
    raise NotImplementedError("write your pallas kernel here")


# speedup vs baseline: 5.2280x; 5.2280x over previous
import functools

import jax
import jax.numpy as jnp
import numpy as np
from jax import lax
from jax.experimental import pallas as pl
from jax.experimental.pallas import tpu as pltpu
from jax.experimental.pallas import tpu_sc as plsc

N_NODE = 1000000
N_DOMAIN = 50000
SIGMA_MAX = float(np.pi)

NW = 32
CHUNK = 128
CPW = 245
NPAD = NW * CPW * CHUNK
NDP = 50048
ROWS = NDP // 128
DSLICE = NDP // 16
TW = 16
AW = 8

_f32 = jnp.float32
_i32 = jnp.int32


def _consts():
    key = jax.random.key(42)
    k_axis, k_ang, k_uni = jax.random.split(key, 3)
    ax = jax.random.normal(k_axis, (N_DOMAIN, 3), dtype=jnp.float32)
    ax = ax / jnp.clip(jnp.linalg.norm(ax, axis=-1, keepdims=True), 1e-12)
    v = jax.random.normal(k_ang, (N_DOMAIN, 3), dtype=jnp.float32)
    n2 = jnp.sum(v * v, axis=-1)
    u = jax.random.uniform(k_uni, (N_DOMAIN,), minval=1e-6, maxval=1.0 - 1e-6)
    t = jnp.clip(jnp.pi * u, 1e-3, jnp.pi - 1e-3)
    for _ in range(25):
        fv = (t - jnp.sin(t)) / jnp.pi - u
        df = (1.0 - jnp.cos(t)) / jnp.pi + 1e-9
        t = jnp.clip(t - fv / df, 0.0, jnp.pi)

    pad = NDP - N_DOMAIN
    axes_c = jnp.pad(ax.T, ((0, 0), (0, pad)))
    axes_c = axes_c.at[0, N_DOMAIN:].set(1.0)
    axes_c = axes_c.reshape(3, ROWS, 128)
    n2_c = jnp.pad(n2, (0, pad)).reshape(ROWS, 128)
    uni_c = jnp.pad(t, (0, pad)).reshape(ROWS, 128)
    zeros_c = jnp.zeros((DSLICE, AW), jnp.float32)
    return axes_c, n2_c, uni_c, zeros_c


def _phase_a(pos_pad, idx_pad, zeros_c):
    mesh = plsc.VectorSubcoreMesh(core_axis_name="c", subcore_axis_name="s")

    @functools.partial(
        pl.kernel,
        mesh=mesh,
        out_type=jax.ShapeDtypeStruct((2 * NDP, AW), jnp.float32),
        scratch_types=[
            pltpu.VMEM((CHUNK, 3), jnp.float32),
            pltpu.VMEM((CHUNK,), jnp.int32),
            pltpu.VMEM((CHUNK, AW), jnp.float32),
            pltpu.VMEM_SHARED((NDP, AW), jnp.float32),
        ],
        compiler_params=pltpu.CompilerParams(
            needs_layout_passes=False, use_tc_tiling_on_sc=False
        ),
    )
    def k(pos_h, idx_h, zero_h, out_h, posv, idxv, buf, acc):
        c = lax.axis_index("c")
        s = lax.axis_index("s")
        w = c * 16 + s
        ids0 = jnp.arange(16, dtype=jnp.int32)

        pltpu.sync_copy(zero_h, acc.at[pl.ds(s * DSLICE, DSLICE)])
        plsc.subcore_barrier()

        for g in range(CHUNK // 16):
            ids = ids0 + g * 16
            plsc.store_scatter(
                buf,
                [ids, jnp.full((16,), 3, jnp.int32)],
                jnp.full((16,), 1.0, jnp.float32),
            )
            for col in range(4, AW):
                plsc.store_scatter(
                    buf,
                    [ids, jnp.full((16,), col, jnp.int32)],
                    jnp.zeros((16,), jnp.float32),
                )

        def body(i, carry):
            base = (w * CPW + i) * CHUNK
            pltpu.sync_copy(pos_h.at[pl.ds(base, CHUNK)], posv)
            pltpu.sync_copy(idx_h.at[pl.ds(base, CHUNK)], idxv)
            for g in range(CHUNK // 16):
                ids = ids0 + g * 16
                for col in range(3):
                    cc = jnp.full((16,), col, jnp.int32)
                    v = plsc.load_gather(posv, [ids, cc])
                    plsc.store_scatter(buf, [ids, cc], v)
            pltpu.sync_copy(buf, acc.at[idxv], add=True)
            return carry

        lax.fori_loop(0, CPW, body, 0)
        plsc.subcore_barrier()
        pltpu.sync_copy(
            acc.at[pl.ds(s * DSLICE, DSLICE)],
            out_h.at[pl.ds(c * NDP + s * DSLICE, DSLICE)],
        )

    return k(pos_pad, idx_pad, zeros_c)


def _phase_b_body(p_ref, ax_ref, n2_ref, uni_ref, info_ref, fp_ref, out_ref):
    sx = p_ref[0, 0] + p_ref[0, 1]
    sy = p_ref[1, 0] + p_ref[1, 1]
    sz = p_ref[2, 0] + p_ref[2, 1]
    cnt = p_ref[3, 0] + p_ref[3, 1]
    inv = 1.0 / jnp.maximum(cnt, 1.0)
    cx = sx * inv
    cy = sy * inv
    cz = sz * inv

    info = info_ref[...]
    sig = SIGMA_MAX * (1.0 - info)
    ang = jnp.clip(jnp.sqrt(sig * sig * n2_ref[...] + 1e-12), 0.0, jnp.pi)
    ang = jnp.where((fp_ref[...] != 0.0) & (info == 0.0), uni_ref[...], ang)
    co = jnp.cos(ang)
    si = jnp.sin(ang)
    omc = 1.0 - co
    ax = ax_ref[0]
    ay = ax_ref[1]
    az = ax_ref[2]

    r00 = co + omc * ax * ax
    r01 = omc * ax * ay - si * az
    r02 = omc * ax * az + si * ay
    r10 = omc * ay * ax + si * az
    r11 = co + omc * ay * ay
    r12 = omc * ay * az - si * ax
    r20 = omc * az * ax - si * ay
    r21 = omc * az * ay + si * ax
    r22 = co + omc * az * az
    t0 = cx - (r00 * cx + r01 * cy + r02 * cz)
    t1 = cy - (r10 * cx + r11 * cy + r12 * cz)
    t2 = cz - (r20 * cx + r21 * cy + r22 * cz)

    vals = [r00, r01, r02, r10, r11, r12, r20, r21, r22, t0, t1, t2]
    for k, v in enumerate(vals):
        out_ref[k] = v
    zero = jnp.zeros_like(r00)
    for k in range(12, 16):
        out_ref[k] = zero


def _phase_b(partials, info_level, from_prior, axes_c, n2_c, uni_c):
    p4 = (
        partials.reshape(2, NDP, AW)[:, :, :4]
        .transpose(2, 0, 1)
        .reshape(4, 2, ROWS, 128)
    )
    info_p = jnp.pad(
        info_level.astype(jnp.float32), (0, NDP - N_DOMAIN), constant_values=1.0
    ).reshape(ROWS, 128)
    fpv = jnp.full((ROWS, 128), jnp.asarray(from_prior, jnp.float32))
    tab16 = pl.pallas_call(
        _phase_b_body,
        out_shape=jax.ShapeDtypeStruct((TW, ROWS, 128), jnp.float32),
    )(p4, axes_c, n2_c, uni_c, info_p, fpv)
    return tab16.reshape(TW, NDP).T


def _phase_c(pos_pad, idx_pad, table):
    mesh = plsc.VectorSubcoreMesh(core_axis_name="c", subcore_axis_name="s")

    @functools.partial(
        pl.kernel,
        mesh=mesh,
        out_type=jax.ShapeDtypeStruct((NPAD, 3), jnp.float32),
        scratch_types=[
            pltpu.VMEM((CHUNK, 3), jnp.float32),
            pltpu.VMEM((CHUNK,), jnp.int32),
            pltpu.VMEM((CHUNK, TW), jnp.float32),
            pltpu.VMEM((CHUNK, 3), jnp.float32),
            pltpu.SemaphoreType.DMA,
        ],
        compiler_params=pltpu.CompilerParams(
            needs_layout_passes=False, use_tc_tiling_on_sc=False
        ),
    )
    def k(pos_h, idx_h, tab_h, out_h, posv, idxv, rowsv, outv, sem):
        c = lax.axis_index("c")
        s = lax.axis_index("s")
        w = c * 16 + s
        ids0 = jnp.arange(16, dtype=jnp.int32)

        def body(i, carry):
            base = (w * CPW + i) * CHUNK
            pltpu.sync_copy(pos_h.at[pl.ds(base, CHUNK)], posv)
            pltpu.sync_copy(idx_h.at[pl.ds(base, CHUNK)], idxv)
            pltpu.async_copy(tab_h.at[idxv], rowsv, sem).wait()
            for g in range(CHUNK // 16):
                ids = ids0 + g * 16

                def col(ref, j):
                    return plsc.load_gather(ref, [ids, jnp.full((16,), j, jnp.int32)])

                x = col(posv, 0)
                y = col(posv, 1)
                z = col(posv, 2)
                r = [col(rowsv, j) for j in range(12)]
                ox = x * r[0] + y * r[1] + z * r[2] + r[9]
                oy = x * r[3] + y * r[4] + z * r[5] + r[10]
                oz = x * r[6] + y * r[7] + z * r[8] + r[11]
                for j, v in ((0, ox), (1, oy), (2, oz)):
                    plsc.store_scatter(
                        outv, [ids, jnp.full((16,), j, jnp.int32)], v
                    )
            pltpu.sync_copy(outv, out_h.at[pl.ds(base, CHUNK)])
            return carry

        lax.fori_loop(0, CPW, body, 0)

    return k(pos_pad, idx_pad, table)


def kernel(pos, info_level, from_prior, domain_node_index):
    axes_c, n2_c, uni_c, zeros_c = _consts()
    pos = pos.astype(jnp.float32)
    didx = domain_node_index[0].astype(jnp.int32)
    n = pos.shape[0]
    pos_pad = jnp.pad(pos, ((0, NPAD - n), (0, 0)))
    idx_pad = jnp.pad(didx, (0, NPAD - n), constant_values=N_DOMAIN)

    partials = _phase_a(pos_pad, idx_pad, zeros_c)
    table = _phase_b(partials, info_level, from_prior, axes_c, n2_c, uni_c)
    out_pad = _phase_c(pos_pad, idx_pad, table)
    return out_pad[:n]


# speedup vs baseline: 5.5513x; 1.0618x over previous
import functools

import jax
import jax.numpy as jnp
import numpy as np
from jax import lax
from jax.experimental import pallas as pl
from jax.experimental.pallas import tpu as pltpu
from jax.experimental.pallas import tpu_sc as plsc

N_NODE = 1000000
N_DOMAIN = 50000
SIGMA_MAX = float(np.pi)

NW = 32
CHUNK = 128
NFULL = N_NODE // CHUNK
TAIL = N_NODE - NFULL * CHUNK
NDP = 50048
ROWS = NDP // 128
DSLICE = NDP // 16
TW = 16
AW = 8

_f32 = jnp.float32
_i32 = jnp.int32


def _consts():
    key = jax.random.key(42)
    k_axis, k_ang, k_uni = jax.random.split(key, 3)
    ax = jax.random.normal(k_axis, (N_DOMAIN, 3), dtype=jnp.float32)
    ax = ax / jnp.clip(jnp.linalg.norm(ax, axis=-1, keepdims=True), 1e-12)
    v = jax.random.normal(k_ang, (N_DOMAIN, 3), dtype=jnp.float32)
    n2 = jnp.sum(v * v, axis=-1)
    u = jax.random.uniform(k_uni, (N_DOMAIN,), minval=1e-6, maxval=1.0 - 1e-6)
    t = jnp.clip(jnp.pi * u, 1e-3, jnp.pi - 1e-3)
    for _ in range(25):
        fv = (t - jnp.sin(t)) / jnp.pi - u
        df = (1.0 - jnp.cos(t)) / jnp.pi + 1e-9
        t = jnp.clip(t - fv / df, 0.0, jnp.pi)

    pad = NDP - N_DOMAIN
    axes_c = jnp.pad(ax.T, ((0, 0), (0, pad)))
    axes_c = axes_c.at[0, N_DOMAIN:].set(1.0)
    axes_c = axes_c.reshape(3, ROWS, 128)
    n2_c = jnp.pad(n2, (0, pad)).reshape(ROWS, 128)
    uni_c = jnp.pad(t, (0, pad)).reshape(ROWS, 128)
    zeros_c = jnp.zeros((DSLICE, AW), jnp.float32)
    return axes_c, n2_c, uni_c, zeros_c


def _repack(posv, buf, ids0, nrows):
    for g in range(nrows // 16):
        ids = ids0 + g * 16
        for col in range(3):
            cc = jnp.full((16,), col, jnp.int32)
            v = plsc.load_gather(posv, [ids, cc])
            plsc.store_scatter(buf, [ids, cc], v)


def _phase_a(pos, dni, zeros_c):
    mesh = plsc.VectorSubcoreMesh(core_axis_name="c", subcore_axis_name="s")

    @functools.partial(
        pl.kernel,
        mesh=mesh,
        out_type=jax.ShapeDtypeStruct((2 * NDP, AW), jnp.float32),
        scratch_types=[
            pltpu.VMEM((CHUNK, 3), jnp.float32),
            pltpu.VMEM((CHUNK,), jnp.int32),
            pltpu.VMEM((CHUNK, AW), jnp.float32),
            pltpu.VMEM_SHARED((NDP, AW), jnp.float32),
        ],
        compiler_params=pltpu.CompilerParams(
            needs_layout_passes=False, use_tc_tiling_on_sc=False
        ),
    )
    def k(pos_h, dni_h, zero_h, out_h, posv, idxv, buf, acc):
        c = lax.axis_index("c")
        s = lax.axis_index("s")
        w = c * 16 + s
        ids0 = jnp.arange(16, dtype=jnp.int32)
        start = w * (NFULL // NW) + jnp.minimum(w, NFULL % NW)
        nt = NFULL // NW + (w < NFULL % NW).astype(jnp.int32)

        pltpu.sync_copy(zero_h, acc.at[pl.ds(s * DSLICE, DSLICE)])
        plsc.subcore_barrier()

        for g in range(CHUNK // 16):
            ids = ids0 + g * 16
            plsc.store_scatter(
                buf,
                [ids, jnp.full((16,), 3, jnp.int32)],
                jnp.full((16,), 1.0, jnp.float32),
            )
            for col in range(4, AW):
                plsc.store_scatter(
                    buf,
                    [ids, jnp.full((16,), col, jnp.int32)],
                    jnp.zeros((16,), jnp.float32),
                )

        def body(i, carry):
            base = (start + i) * CHUNK
            pltpu.sync_copy(pos_h.at[pl.ds(base, CHUNK)], posv)
            pltpu.sync_copy(dni_h.at[0, pl.ds(base, CHUNK)], idxv)
            _repack(posv, buf, ids0, CHUNK)
            pltpu.sync_copy(buf, acc.at[idxv], add=True)
            return carry

        lax.fori_loop(0, nt, body, 0)

        @pl.when(w == NW - 1)
        def _():
            base = NFULL * CHUNK
            pltpu.sync_copy(
                pos_h.at[pl.ds(base, TAIL)], posv.at[pl.ds(0, TAIL)]
            )
            pltpu.sync_copy(
                dni_h.at[0, pl.ds(base, TAIL)], idxv.at[pl.ds(0, TAIL)]
            )
            _repack(posv, buf, ids0, TAIL)
            for g in range(TAIL // 16, CHUNK // 16):
                ids = ids0 + g * 16
                for col in range(4):
                    plsc.store_scatter(
                        buf,
                        [ids, jnp.full((16,), col, jnp.int32)],
                        jnp.zeros((16,), jnp.float32),
                    )
            pltpu.sync_copy(buf, acc.at[idxv], add=True)

        plsc.subcore_barrier()
        pltpu.sync_copy(
            acc.at[pl.ds(s * DSLICE, DSLICE)],
            out_h.at[pl.ds(c * NDP + s * DSLICE, DSLICE)],
        )

    return k(pos, dni, zeros_c)


def _phase_b_body(p_ref, ax_ref, n2_ref, uni_ref, info_ref, fp_ref, out_ref):
    sx = p_ref[0, 0] + p_ref[0, 1]
    sy = p_ref[1, 0] + p_ref[1, 1]
    sz = p_ref[2, 0] + p_ref[2, 1]
    cnt = p_ref[3, 0] + p_ref[3, 1]
    inv = 1.0 / jnp.maximum(cnt, 1.0)
    cx = sx * inv
    cy = sy * inv
    cz = sz * inv

    info = info_ref[...]
    sig = SIGMA_MAX * (1.0 - info)
    ang = jnp.clip(jnp.sqrt(sig * sig * n2_ref[...] + 1e-12), 0.0, jnp.pi)
    ang = jnp.where((fp_ref[...] != 0.0) & (info == 0.0), uni_ref[...], ang)
    co = jnp.cos(ang)
    si = jnp.sin(ang)
    omc = 1.0 - co
    ax = ax_ref[0]
    ay = ax_ref[1]
    az = ax_ref[2]

    r00 = co + omc * ax * ax
    r01 = omc * ax * ay - si * az
    r02 = omc * ax * az + si * ay
    r10 = omc * ay * ax + si * az
    r11 = co + omc * ay * ay
    r12 = omc * ay * az - si * ax
    r20 = omc * az * ax - si * ay
    r21 = omc * az * ay + si * ax
    r22 = co + omc * az * az
    t0 = cx - (r00 * cx + r01 * cy + r02 * cz)
    t1 = cy - (r10 * cx + r11 * cy + r12 * cz)
    t2 = cz - (r20 * cx + r21 * cy + r22 * cz)

    vals = [r00, r01, r02, r10, r11, r12, r20, r21, r22, t0, t1, t2]
    for k, v in enumerate(vals):
        out_ref[k] = v
    zero = jnp.zeros_like(r00)
    for k in range(12, 16):
        out_ref[k] = zero


def _phase_b(partials, info_level, from_prior, axes_c, n2_c, uni_c):
    p4 = (
        partials.reshape(2, NDP, AW)[:, :, :4]
        .transpose(2, 0, 1)
        .reshape(4, 2, ROWS, 128)
    )
    info_p = jnp.pad(
        info_level.astype(jnp.float32), (0, NDP - N_DOMAIN), constant_values=1.0
    ).reshape(ROWS, 128)
    fpv = jnp.full((ROWS, 128), jnp.asarray(from_prior, jnp.float32))
    tab16 = pl.pallas_call(
        _phase_b_body,
        out_shape=jax.ShapeDtypeStruct((TW, ROWS, 128), jnp.float32),
    )(p4, axes_c, n2_c, uni_c, info_p, fpv)
    return tab16.reshape(TW, NDP).T


def _phase_c(pos, dni, table):
    nslot = NFULL + 1
    mesh = plsc.VectorSubcoreMesh(core_axis_name="c", subcore_axis_name="s")

    @functools.partial(
        pl.kernel,
        mesh=mesh,
        out_type=jax.ShapeDtypeStruct((N_NODE, 3), jnp.float32),
        scratch_types=[
            pltpu.VMEM((CHUNK, 3), jnp.float32),
            pltpu.VMEM((CHUNK,), jnp.int32),
            pltpu.VMEM((CHUNK, TW), jnp.float32),
            pltpu.VMEM((CHUNK, 3), jnp.float32),
            pltpu.SemaphoreType.DMA,
        ],
        compiler_params=pltpu.CompilerParams(
            needs_layout_passes=False, use_tc_tiling_on_sc=False
        ),
    )
    def k(pos_h, dni_h, tab_h, out_h, posv, idxv, rowsv, outv, sem):
        c = lax.axis_index("c")
        s = lax.axis_index("s")
        w = c * 16 + s
        ids0 = jnp.arange(16, dtype=jnp.int32)
        start = w * (nslot // NW) + jnp.minimum(w, nslot % NW)
        nt = nslot // NW + (w < nslot % NW).astype(jnp.int32)

        def body(i, carry):
            base = jnp.minimum((start + i) * CHUNK, N_NODE - CHUNK)
            pltpu.sync_copy(pos_h.at[pl.ds(base, CHUNK)], posv)
            pltpu.sync_copy(dni_h.at[0, pl.ds(base, CHUNK)], idxv)
            pltpu.async_copy(tab_h.at[idxv], rowsv, sem).wait()
            for g in range(CHUNK // 16):
                ids = ids0 + g * 16

                def col(ref, j):
                    return plsc.load_gather(ref, [ids, jnp.full((16,), j, jnp.int32)])

                x = col(posv, 0)
                y = col(posv, 1)
                z = col(posv, 2)
                r = [col(rowsv, j) for j in range(12)]
                ox = x * r[0] + y * r[1] + z * r[2] + r[9]
                oy = x * r[3] + y * r[4] + z * r[5] + r[10]
                oz = x * r[6] + y * r[7] + z * r[8] + r[11]
                for j, v in ((0, ox), (1, oy), (2, oz)):
                    plsc.store_scatter(
                        outv, [ids, jnp.full((16,), j, jnp.int32)], v
                    )
            pltpu.sync_copy(outv, out_h.at[pl.ds(base, CHUNK)])
            return carry

        lax.fori_loop(0, nt, body, 0)

    return k(pos, dni, table)


def kernel(pos, info_level, from_prior, domain_node_index):
    axes_c, n2_c, uni_c, zeros_c = _consts()
    pos = pos.astype(jnp.float32)
    dni = domain_node_index.astype(jnp.int32)

    partials = _phase_a(pos, dni, zeros_c)
    table = _phase_b(partials, info_level, from_prior, axes_c, n2_c, uni_c)
    return _phase_c(pos, dni, table)


# speedup vs baseline: 18.5692x; 3.3450x over previous
import functools

import jax
import jax.numpy as jnp
import numpy as np
from jax import lax
from jax.experimental import pallas as pl
from jax.experimental.pallas import tpu as pltpu
from jax.experimental.pallas import tpu_sc as plsc

N_NODE = 1000000
N_DOMAIN = 50000
SIGMA_MAX = float(np.pi)

NW = 32
CHUNK = 128
NFULL = N_NODE // CHUNK
TAIL = N_NODE - NFULL * CHUNK
NDP = 50048
ROWS = NDP // 128
DSLICE = NDP // 16
TW = 16
AW = 8

_f32 = jnp.float32
_i32 = jnp.int32


def _consts():
    key = jax.random.key(42)
    k_axis, k_ang, k_uni = jax.random.split(key, 3)
    ax = jax.random.normal(k_axis, (N_DOMAIN, 3), dtype=jnp.float32)
    ax = ax / jnp.clip(jnp.linalg.norm(ax, axis=-1, keepdims=True), 1e-12)
    v = jax.random.normal(k_ang, (N_DOMAIN, 3), dtype=jnp.float32)
    n2 = jnp.sum(v * v, axis=-1)
    u = jax.random.uniform(k_uni, (N_DOMAIN,), minval=1e-6, maxval=1.0 - 1e-6)
    t = jnp.clip(jnp.pi * u, 1e-3, jnp.pi - 1e-3)
    for _ in range(25):
        fv = (t - jnp.sin(t)) / jnp.pi - u
        df = (1.0 - jnp.cos(t)) / jnp.pi + 1e-9
        t = jnp.clip(t - fv / df, 0.0, jnp.pi)

    pad = NDP - N_DOMAIN
    axes_c = jnp.pad(ax.T, ((0, 0), (0, pad)))
    axes_c = axes_c.at[0, N_DOMAIN:].set(1.0)
    axes_c = axes_c.reshape(3, ROWS, 128)
    n2_c = jnp.pad(n2, (0, pad)).reshape(ROWS, 128)
    uni_c = jnp.pad(t, (0, pad)).reshape(ROWS, 128)
    zeros_c = jnp.zeros((DSLICE, AW), jnp.float32)
    return axes_c, n2_c, uni_c, zeros_c


def _repack(pxyz, buf, ids0, nrows):
    for g in range(nrows // 16):
        ids = ids0 + g * 16
        for col in range(3):
            cc = jnp.full((16,), col, jnp.int32)
            v = pxyz[col][pl.ds(g * 16, 16)]
            plsc.store_scatter(buf, [ids, cc], v)


def _phase_a(pos_t, didx, zeros_c):
    mesh = plsc.VectorSubcoreMesh(core_axis_name="c", subcore_axis_name="s")

    @functools.partial(
        pl.kernel,
        mesh=mesh,
        out_type=jax.ShapeDtypeStruct((AW, 2 * NDP), jnp.float32),
        scratch_types=[
            pltpu.VMEM((CHUNK,), jnp.float32),
            pltpu.VMEM((CHUNK,), jnp.float32),
            pltpu.VMEM((CHUNK,), jnp.float32),
            pltpu.VMEM((CHUNK,), jnp.int32),
            pltpu.VMEM((CHUNK, AW), jnp.float32),
            pltpu.VMEM_SHARED((NDP, AW), jnp.float32),
            pltpu.VMEM((DSLICE, AW), jnp.float32),
            pltpu.VMEM((AW, DSLICE), jnp.float32),
        ],
        compiler_params=pltpu.CompilerParams(
            needs_layout_passes=False, use_tc_tiling_on_sc=False
        ),
    )
    def k(pos_h, idx_h, zero_h, out_h, pxv, pyv, pzv, idxv, buf, acc, tin, tout):
        c = lax.axis_index("c")
        s = lax.axis_index("s")
        w = c * 16 + s
        ids0 = jnp.arange(16, dtype=jnp.int32)
        start = w * (NFULL // NW) + jnp.minimum(w, NFULL % NW)
        nt = NFULL // NW + (w < NFULL % NW).astype(jnp.int32)

        pltpu.sync_copy(zero_h, acc.at[pl.ds(s * DSLICE, DSLICE)])
        plsc.subcore_barrier()

        for g in range(CHUNK // 16):
            ids = ids0 + g * 16
            plsc.store_scatter(
                buf,
                [ids, jnp.full((16,), 3, jnp.int32)],
                jnp.full((16,), 1.0, jnp.float32),
            )
            for col in range(4, AW):
                plsc.store_scatter(
                    buf,
                    [ids, jnp.full((16,), col, jnp.int32)],
                    jnp.zeros((16,), jnp.float32),
                )

        def body(i, carry):
            base = (start + i) * CHUNK
            pltpu.sync_copy(pos_h.at[0, pl.ds(base, CHUNK)], pxv)
            pltpu.sync_copy(pos_h.at[1, pl.ds(base, CHUNK)], pyv)
            pltpu.sync_copy(pos_h.at[2, pl.ds(base, CHUNK)], pzv)
            pltpu.sync_copy(idx_h.at[pl.ds(base, CHUNK)], idxv)
            _repack((pxv, pyv, pzv), buf, ids0, CHUNK)
            pltpu.sync_copy(buf, acc.at[idxv], add=True)
            return carry

        lax.fori_loop(0, nt, body, 0)

        @pl.when(w == NW - 1)
        def _():
            base = NFULL * CHUNK
            pltpu.sync_copy(pos_h.at[0, pl.ds(base, TAIL)], pxv.at[pl.ds(0, TAIL)])
            pltpu.sync_copy(pos_h.at[1, pl.ds(base, TAIL)], pyv.at[pl.ds(0, TAIL)])
            pltpu.sync_copy(pos_h.at[2, pl.ds(base, TAIL)], pzv.at[pl.ds(0, TAIL)])
            pltpu.sync_copy(idx_h.at[pl.ds(base, TAIL)], idxv.at[pl.ds(0, TAIL)])
            _repack((pxv, pyv, pzv), buf, ids0, TAIL)
            for g in range(TAIL // 16, CHUNK // 16):
                ids = ids0 + g * 16
                for col in range(4):
                    plsc.store_scatter(
                        buf,
                        [ids, jnp.full((16,), col, jnp.int32)],
                        jnp.zeros((16,), jnp.float32),
                    )
            pltpu.sync_copy(buf, acc.at[idxv], add=True)

        plsc.subcore_barrier()

        pltpu.sync_copy(acc.at[pl.ds(s * DSLICE, DSLICE)], tin)

        def tbody(i, carry):
            ids = ids0 + i * 16
            for col in range(AW):
                v = plsc.load_gather(tin, [ids, jnp.full((16,), col, jnp.int32)])
                tout[col, pl.ds(i * 16, 16)] = v
            return carry

        lax.fori_loop(0, DSLICE // 16, tbody, 0)
        for col in range(AW):
            pltpu.sync_copy(
                tout.at[col],
                out_h.at[col, pl.ds(c * NDP + s * DSLICE, DSLICE)],
            )

    return k(pos_t, didx, zeros_c)


def _phase_b_body(p_ref, ax_ref, n2_ref, uni_ref, info_ref, fp_ref, out_ref):
    sx = p_ref[0, 0] + p_ref[0, 1]
    sy = p_ref[1, 0] + p_ref[1, 1]
    sz = p_ref[2, 0] + p_ref[2, 1]
    cnt = p_ref[3, 0] + p_ref[3, 1]
    inv = 1.0 / jnp.maximum(cnt, 1.0)
    cx = sx * inv
    cy = sy * inv
    cz = sz * inv

    info = info_ref[...]
    sig = SIGMA_MAX * (1.0 - info)
    ang = jnp.clip(jnp.sqrt(sig * sig * n2_ref[...] + 1e-12), 0.0, jnp.pi)
    ang = jnp.where((fp_ref[...] != 0.0) & (info == 0.0), uni_ref[...], ang)
    co = jnp.cos(ang)
    si = jnp.sin(ang)
    omc = 1.0 - co
    ax = ax_ref[0]
    ay = ax_ref[1]
    az = ax_ref[2]

    r00 = co + omc * ax * ax
    r01 = omc * ax * ay - si * az
    r02 = omc * ax * az + si * ay
    r10 = omc * ay * ax + si * az
    r11 = co + omc * ay * ay
    r12 = omc * ay * az - si * ax
    r20 = omc * az * ax - si * ay
    r21 = omc * az * ay + si * ax
    r22 = co + omc * az * az
    t0 = cx - (r00 * cx + r01 * cy + r02 * cz)
    t1 = cy - (r10 * cx + r11 * cy + r12 * cz)
    t2 = cz - (r20 * cx + r21 * cy + r22 * cz)

    vals = [r00, r01, r02, r10, r11, r12, r20, r21, r22, t0, t1, t2]
    for k, v in enumerate(vals):
        out_ref[k] = v
    zero = jnp.zeros_like(r00)
    for k in range(12, 16):
        out_ref[k] = zero


def _phase_b(partials, info_level, from_prior, axes_c, n2_c, uni_c):
    p4 = partials.reshape(AW, 2, ROWS, 128)[:4]
    info_p = jnp.pad(
        info_level.astype(jnp.float32), (0, NDP - N_DOMAIN), constant_values=1.0
    ).reshape(ROWS, 128)
    fpv = jnp.full((ROWS, 128), jnp.asarray(from_prior, jnp.float32))
    tab16 = pl.pallas_call(
        _phase_b_body,
        out_shape=jax.ShapeDtypeStruct((TW, ROWS, 128), jnp.float32),
    )(p4, axes_c, n2_c, uni_c, info_p, fpv)
    return tab16.reshape(TW, NDP).T


def _phase_c(pos_t, didx, table):
    nslot = NFULL + 1
    mesh = plsc.VectorSubcoreMesh(core_axis_name="c", subcore_axis_name="s")

    @functools.partial(
        pl.kernel,
        mesh=mesh,
        out_type=jax.ShapeDtypeStruct((3, N_NODE), jnp.float32),
        scratch_types=[
            pltpu.VMEM((CHUNK,), jnp.float32),
            pltpu.VMEM((CHUNK,), jnp.float32),
            pltpu.VMEM((CHUNK,), jnp.float32),
            pltpu.VMEM((CHUNK,), jnp.int32),
            pltpu.VMEM((CHUNK, TW), jnp.float32),
            pltpu.VMEM((CHUNK,), jnp.float32),
            pltpu.VMEM((CHUNK,), jnp.float32),
            pltpu.VMEM((CHUNK,), jnp.float32),
            pltpu.SemaphoreType.DMA,
        ],
        compiler_params=pltpu.CompilerParams(
            needs_layout_passes=False, use_tc_tiling_on_sc=False
        ),
    )
    def k(pos_h, idx_h, tab_h, out_h, pxv, pyv, pzv, idxv, rowsv, oxv, oyv, ozv, sem):
        c = lax.axis_index("c")
        s = lax.axis_index("s")
        w = c * 16 + s
        ids0 = jnp.arange(16, dtype=jnp.int32)
        start = w * (nslot // NW) + jnp.minimum(w, nslot % NW)
        nt = nslot // NW + (w < nslot % NW).astype(jnp.int32)

        def body(i, carry):
            base = jnp.minimum((start + i) * CHUNK, N_NODE - CHUNK)
            pltpu.sync_copy(pos_h.at[0, pl.ds(base, CHUNK)], pxv)
            pltpu.sync_copy(pos_h.at[1, pl.ds(base, CHUNK)], pyv)
            pltpu.sync_copy(pos_h.at[2, pl.ds(base, CHUNK)], pzv)
            pltpu.sync_copy(idx_h.at[pl.ds(base, CHUNK)], idxv)
            pltpu.async_copy(tab_h.at[idxv], rowsv, sem).wait()
            for g in range(CHUNK // 16):
                ids = ids0 + g * 16

                def col(j):
                    return plsc.load_gather(
                        rowsv, [ids, jnp.full((16,), j, jnp.int32)]
                    )

                x = pxv[pl.ds(g * 16, 16)]
                y = pyv[pl.ds(g * 16, 16)]
                z = pzv[pl.ds(g * 16, 16)]
                r = [col(j) for j in range(12)]
                oxv[pl.ds(g * 16, 16)] = x * r[0] + y * r[1] + z * r[2] + r[9]
                oyv[pl.ds(g * 16, 16)] = x * r[3] + y * r[4] + z * r[5] + r[10]
                ozv[pl.ds(g * 16, 16)] = x * r[6] + y * r[7] + z * r[8] + r[11]
            pltpu.sync_copy(oxv, out_h.at[0, pl.ds(base, CHUNK)])
            pltpu.sync_copy(oyv, out_h.at[1, pl.ds(base, CHUNK)])
            pltpu.sync_copy(ozv, out_h.at[2, pl.ds(base, CHUNK)])
            return carry

        lax.fori_loop(0, nt, body, 0)

    return k(pos_t, didx, table)


def kernel(pos, info_level, from_prior, domain_node_index):
    axes_c, n2_c, uni_c, zeros_c = _consts()
    pos_t = pos.astype(jnp.float32).T
    didx = domain_node_index[0].astype(jnp.int32)

    partials = _phase_a(pos_t, didx, zeros_c)
    table = _phase_b(partials, info_level, from_prior, axes_c, n2_c, uni_c)
    return _phase_c(pos_t, didx, table).T


# speedup vs baseline: 18.5908x; 1.0012x over previous
import functools

import jax
import jax.numpy as jnp
import numpy as np
from jax import lax
from jax.experimental import pallas as pl
from jax.experimental.pallas import tpu as pltpu
from jax.experimental.pallas import tpu_sc as plsc

N_NODE = 1000000
N_DOMAIN = 50000
SIGMA_MAX = float(np.pi)

NW = 32
CHUNK = 128
NFULL = N_NODE // CHUNK
TAIL = N_NODE - NFULL * CHUNK
NDP = 50176
ROWS = NDP // 128
DSLICE = NDP // 16
TW = 16
AW = 8

_f32 = jnp.float32
_i32 = jnp.int32


def _consts():
    key = jax.random.key(42)
    k_axis, k_ang, k_uni = jax.random.split(key, 3)
    ax = jax.random.normal(k_axis, (N_DOMAIN, 3), dtype=jnp.float32)
    ax = ax / jnp.clip(jnp.linalg.norm(ax, axis=-1, keepdims=True), 1e-12)
    v = jax.random.normal(k_ang, (N_DOMAIN, 3), dtype=jnp.float32)
    n2 = jnp.sum(v * v, axis=-1)
    u = jax.random.uniform(k_uni, (N_DOMAIN,), minval=1e-6, maxval=1.0 - 1e-6)
    t = jnp.clip(jnp.pi * u, 1e-3, jnp.pi - 1e-3)
    for _ in range(25):
        fv = (t - jnp.sin(t)) / jnp.pi - u
        df = (1.0 - jnp.cos(t)) / jnp.pi + 1e-9
        t = jnp.clip(t - fv / df, 0.0, jnp.pi)

    pad = NDP - N_DOMAIN
    axes_c = jnp.pad(ax.T, ((0, 0), (0, pad)))
    axes_c = axes_c.at[0, N_DOMAIN:].set(1.0)
    axes_c = axes_c.reshape(3, ROWS, 128)
    n2_c = jnp.pad(n2, (0, pad)).reshape(ROWS, 128)
    uni_c = jnp.pad(t, (0, pad)).reshape(ROWS, 128)
    zeros_c = jnp.zeros((DSLICE, AW), jnp.float32)
    return axes_c, n2_c, uni_c, zeros_c


def _repack(pxyz, buf, ids0, nrows):
    for g in range(nrows // 16):
        ids = ids0 + g * 16
        for col in range(3):
            cc = jnp.full((16,), col, jnp.int32)
            v = pxyz[col][pl.ds(g * 16, 16)]
            plsc.store_scatter(buf, [ids, cc], v)


def _phase_a(pos_t, didx, zeros_c):
    mesh = plsc.VectorSubcoreMesh(core_axis_name="c", subcore_axis_name="s")

    @functools.partial(
        pl.kernel,
        mesh=mesh,
        out_type=jax.ShapeDtypeStruct((AW, 2 * NDP), jnp.float32),
        scratch_types=[
            pltpu.VMEM((CHUNK,), jnp.float32),
            pltpu.VMEM((CHUNK,), jnp.float32),
            pltpu.VMEM((CHUNK,), jnp.float32),
            pltpu.VMEM((CHUNK,), jnp.int32),
            pltpu.VMEM((CHUNK, AW), jnp.float32),
            pltpu.VMEM_SHARED((NDP, AW), jnp.float32),
            pltpu.VMEM((DSLICE, AW), jnp.float32),
            pltpu.VMEM((AW, DSLICE), jnp.float32),
        ],
        compiler_params=pltpu.CompilerParams(
            needs_layout_passes=False, use_tc_tiling_on_sc=False
        ),
    )
    def k(pos_h, idx_h, zero_h, out_h, pxv, pyv, pzv, idxv, buf, acc, tin, tout):
        c = lax.axis_index("c")
        s = lax.axis_index("s")
        w = c * 16 + s
        ids0 = jnp.arange(16, dtype=jnp.int32)
        start = w * (NFULL // NW) + jnp.minimum(w, NFULL % NW)
        nt = NFULL // NW + (w < NFULL % NW).astype(jnp.int32)

        pltpu.sync_copy(zero_h, acc.at[pl.ds(s * DSLICE, DSLICE)])
        plsc.subcore_barrier()

        for g in range(CHUNK // 16):
            ids = ids0 + g * 16
            plsc.store_scatter(
                buf,
                [ids, jnp.full((16,), 3, jnp.int32)],
                jnp.full((16,), 1.0, jnp.float32),
            )
            for col in range(4, AW):
                plsc.store_scatter(
                    buf,
                    [ids, jnp.full((16,), col, jnp.int32)],
                    jnp.zeros((16,), jnp.float32),
                )

        def body(i, carry):
            base = (start + i) * CHUNK
            pltpu.sync_copy(pos_h.at[0, pl.ds(base, CHUNK)], pxv)
            pltpu.sync_copy(pos_h.at[1, pl.ds(base, CHUNK)], pyv)
            pltpu.sync_copy(pos_h.at[2, pl.ds(base, CHUNK)], pzv)
            pltpu.sync_copy(idx_h.at[pl.ds(base, CHUNK)], idxv)
            _repack((pxv, pyv, pzv), buf, ids0, CHUNK)
            pltpu.sync_copy(buf, acc.at[idxv], add=True)
            return carry

        lax.fori_loop(0, nt, body, 0)

        @pl.when(w == NW - 1)
        def _():
            base = NFULL * CHUNK
            pltpu.sync_copy(pos_h.at[0, pl.ds(base, TAIL)], pxv.at[pl.ds(0, TAIL)])
            pltpu.sync_copy(pos_h.at[1, pl.ds(base, TAIL)], pyv.at[pl.ds(0, TAIL)])
            pltpu.sync_copy(pos_h.at[2, pl.ds(base, TAIL)], pzv.at[pl.ds(0, TAIL)])
            pltpu.sync_copy(idx_h.at[pl.ds(base, TAIL)], idxv.at[pl.ds(0, TAIL)])
            _repack((pxv, pyv, pzv), buf, ids0, TAIL)
            for g in range(TAIL // 16, CHUNK // 16):
                ids = ids0 + g * 16
                for col in range(4):
                    plsc.store_scatter(
                        buf,
                        [ids, jnp.full((16,), col, jnp.int32)],
                        jnp.zeros((16,), jnp.float32),
                    )
            pltpu.sync_copy(buf, acc.at[idxv], add=True)

        plsc.subcore_barrier()

        pltpu.sync_copy(acc.at[pl.ds(s * DSLICE, DSLICE)], tin)

        def tbody(i, carry):
            ids = ids0 + i * 16
            for col in range(AW):
                v = plsc.load_gather(tin, [ids, jnp.full((16,), col, jnp.int32)])
                tout[col, pl.ds(i * 16, 16)] = v
            return carry

        lax.fori_loop(0, DSLICE // 16, tbody, 0)
        for col in range(AW):
            pltpu.sync_copy(
                tout.at[col],
                out_h.at[col, pl.ds(c * NDP + s * DSLICE, DSLICE)],
            )

    return k(pos_t, didx, zeros_c)


def _phase_b_body(p_ref, ax_ref, n2_ref, uni_ref, info_ref, fp_ref, out_ref):
    sx = p_ref[0, 0] + p_ref[0, 1]
    sy = p_ref[1, 0] + p_ref[1, 1]
    sz = p_ref[2, 0] + p_ref[2, 1]
    cnt = p_ref[3, 0] + p_ref[3, 1]
    inv = 1.0 / jnp.maximum(cnt, 1.0)
    cx = sx * inv
    cy = sy * inv
    cz = sz * inv

    info = info_ref[...]
    sig = SIGMA_MAX * (1.0 - info)
    ang = jnp.clip(jnp.sqrt(sig * sig * n2_ref[...] + 1e-12), 0.0, jnp.pi)
    ang = jnp.where((fp_ref[...] != 0.0) & (info == 0.0), uni_ref[...], ang)
    co = jnp.cos(ang)
    si = jnp.sin(ang)
    omc = 1.0 - co
    ax = ax_ref[0]
    ay = ax_ref[1]
    az = ax_ref[2]

    r00 = co + omc * ax * ax
    r01 = omc * ax * ay - si * az
    r02 = omc * ax * az + si * ay
    r10 = omc * ay * ax + si * az
    r11 = co + omc * ay * ay
    r12 = omc * ay * az - si * ax
    r20 = omc * az * ax - si * ay
    r21 = omc * az * ay + si * ax
    r22 = co + omc * az * az
    t0 = cx - (r00 * cx + r01 * cy + r02 * cz)
    t1 = cy - (r10 * cx + r11 * cy + r12 * cz)
    t2 = cz - (r20 * cx + r21 * cy + r22 * cz)

    vals = [r00, r01, r02, r10, r11, r12, r20, r21, r22, t0, t1, t2]
    for k, v in enumerate(vals):
        out_ref[k] = v
    zero = jnp.zeros_like(r00)
    for k in range(12, 16):
        out_ref[k] = zero


def _phase_b(partials, info_level, from_prior, axes_c, n2_c, uni_c):
    p4 = partials.reshape(AW, 2, ROWS, 128)[:4]
    info_p = jnp.pad(
        info_level.astype(jnp.float32), (0, NDP - N_DOMAIN), constant_values=1.0
    ).reshape(ROWS, 128)
    fpv = jnp.full((ROWS, 128), jnp.asarray(from_prior, jnp.float32))
    tab16 = pl.pallas_call(
        _phase_b_body,
        out_shape=jax.ShapeDtypeStruct((TW, ROWS, 128), jnp.float32),
    )(p4, axes_c, n2_c, uni_c, info_p, fpv)
    return tab16.reshape(TW, NDP).T


def _phase_c(pos_t, didx, table):
    nslot = NFULL + 1
    mesh = plsc.VectorSubcoreMesh(core_axis_name="c", subcore_axis_name="s")

    @functools.partial(
        pl.kernel,
        mesh=mesh,
        out_type=jax.ShapeDtypeStruct((3, N_NODE), jnp.float32),
        scratch_types=[
            pltpu.VMEM((CHUNK,), jnp.float32),
            pltpu.VMEM((CHUNK,), jnp.float32),
            pltpu.VMEM((CHUNK,), jnp.float32),
            pltpu.VMEM((CHUNK,), jnp.int32),
            pltpu.VMEM((CHUNK, TW), jnp.float32),
            pltpu.VMEM((CHUNK,), jnp.float32),
            pltpu.VMEM((CHUNK,), jnp.float32),
            pltpu.VMEM((CHUNK,), jnp.float32),
            pltpu.SemaphoreType.DMA,
        ],
        compiler_params=pltpu.CompilerParams(
            needs_layout_passes=False, use_tc_tiling_on_sc=False
        ),
    )
    def k(pos_h, idx_h, tab_h, out_h, pxv, pyv, pzv, idxv, rowsv, oxv, oyv, ozv, sem):
        c = lax.axis_index("c")
        s = lax.axis_index("s")
        w = c * 16 + s
        ids0 = jnp.arange(16, dtype=jnp.int32)
        start = w * (nslot // NW) + jnp.minimum(w, nslot % NW)
        nt = nslot // NW + (w < nslot % NW).astype(jnp.int32)

        def body(i, carry):
            base = jnp.minimum((start + i) * CHUNK, N_NODE - CHUNK)
            pltpu.sync_copy(pos_h.at[0, pl.ds(base, CHUNK)], pxv)
            pltpu.sync_copy(pos_h.at[1, pl.ds(base, CHUNK)], pyv)
            pltpu.sync_copy(pos_h.at[2, pl.ds(base, CHUNK)], pzv)
            pltpu.sync_copy(idx_h.at[pl.ds(base, CHUNK)], idxv)
            pltpu.async_copy(tab_h.at[idxv], rowsv, sem).wait()
            for g in range(CHUNK // 16):
                ids = ids0 + g * 16

                def col(j):
                    return plsc.load_gather(
                        rowsv, [ids, jnp.full((16,), j, jnp.int32)]
                    )

                x = pxv[pl.ds(g * 16, 16)]
                y = pyv[pl.ds(g * 16, 16)]
                z = pzv[pl.ds(g * 16, 16)]
                r = [col(j) for j in range(12)]
                oxv[pl.ds(g * 16, 16)] = x * r[0] + y * r[1] + z * r[2] + r[9]
                oyv[pl.ds(g * 16, 16)] = x * r[3] + y * r[4] + z * r[5] + r[10]
                ozv[pl.ds(g * 16, 16)] = x * r[6] + y * r[7] + z * r[8] + r[11]
            pltpu.sync_copy(oxv, out_h.at[0, pl.ds(base, CHUNK)])
            pltpu.sync_copy(oyv, out_h.at[1, pl.ds(base, CHUNK)])
            pltpu.sync_copy(ozv, out_h.at[2, pl.ds(base, CHUNK)])
            return carry

        lax.fori_loop(0, nt, body, 0)

    return k(pos_t, didx, table)


def kernel(pos, info_level, from_prior, domain_node_index):
    axes_c, n2_c, uni_c, zeros_c = _consts()
    pos_t = pos.astype(jnp.float32).T
    didx = domain_node_index[0].astype(jnp.int32)

    partials = _phase_a(pos_t, didx, zeros_c)
    table = _phase_b(partials, info_level, from_prior, axes_c, n2_c, uni_c)
    return _phase_c(pos_t, didx, table).T
